# Initial kernel scaffold; baseline (speedup 1.0000x reference)
import jax
import jax.numpy as jnp
from jax.experimental import pallas as pl


def kernel(encoded_data, embedding_table):
    raise NotImplementedError("write your pallas kernel here")


# speedup vs baseline: 1.2925x; 1.2925x over previous
import functools
import math

import jax
import jax.numpy as jnp
from jax import lax
from jax.experimental import pallas as pl
from jax.experimental.pallas import tpu as pltpu
from jax.experimental.pallas import tpu_sc as plsc

_D = 32
_L = 16
_NC, _NS = 2, 16
_NW = _NC * _NS
_CHUNK = 1024
_SCALE = math.sqrt(32.0)


def kernel(encoded_data, embedding_table):
    batch, seqlen = encoded_data.shape
    n = batch * seqlen
    per_w = n // _NW
    nchunk = per_w // _CHUNK
    assert per_w * _NW == n and nchunk * _CHUNK == per_w

    idx_flat = encoded_data.reshape(n).astype(jnp.int32)

    mesh = plsc.VectorSubcoreMesh(
        core_axis_name="c", subcore_axis_name="s",
        num_cores=_NC, num_subcores=_NS)

    @functools.partial(
        pl.kernel,
        out_type=jax.ShapeDtypeStruct((n, _D), jnp.float32),
        mesh=mesh,
        scratch_types=[
            pltpu.VMEM((_CHUNK,), jnp.int32),
            pltpu.VMEM((_CHUNK, _D), jnp.float32),
            pltpu.SemaphoreType.DMA,
        ],
        compiler_params=pltpu.CompilerParams(use_tc_tiling_on_sc=False),
    )
    def emb(idx_hbm, table_hbm, out_hbm, idx_v, rows_v, sem):
        wid = lax.axis_index("s") * _NC + lax.axis_index("c")
        w_base = wid * per_w

        def chunk_body(c, carry):
            base = w_base + c * _CHUNK
            pltpu.sync_copy(idx_hbm.at[pl.ds(base, _CHUNK)], idx_v)
            pltpu.async_copy(table_hbm.at[idx_v], rows_v, sem).wait()

            def scale_row(i, carry2):
                rows_v[i, pl.ds(0, _L)] = rows_v[i, pl.ds(0, _L)] * _SCALE
                rows_v[i, pl.ds(_L, _L)] = rows_v[i, pl.ds(_L, _L)] * _SCALE
                return carry2

            lax.fori_loop(0, _CHUNK, scale_row, 0)
            pltpu.sync_copy(rows_v, out_hbm.at[pl.ds(base, _CHUNK)])
            return carry

        lax.fori_loop(0, nchunk, chunk_body, 0)

    out = emb(idx_flat, embedding_table)
    return out.reshape(batch, seqlen, _D)


# speedup vs baseline: 1.3721x; 1.0616x over previous
import functools
import math

import jax
import jax.numpy as jnp
from jax import lax
from jax.experimental import pallas as pl
from jax.experimental.pallas import tpu as pltpu
from jax.experimental.pallas import tpu_sc as plsc

_D = 32
_L = 16
_NC, _NS = 2, 16
_NW = _NC * _NS
_CHUNK = 1280
_NBUF = 2
_SCALE = math.sqrt(32.0)


def kernel(encoded_data, embedding_table):
    batch, seqlen = encoded_data.shape
    n = batch * seqlen
    per_w = n // _NW
    nchunk = per_w // _CHUNK
    nsuper = nchunk // _NBUF
    assert per_w * _NW == n and nchunk * _CHUNK == per_w
    assert nsuper * _NBUF == nchunk and nsuper >= 2

    idx_flat = encoded_data.reshape(n).astype(jnp.int32)

    mesh = plsc.VectorSubcoreMesh(
        core_axis_name="c", subcore_axis_name="s",
        num_cores=_NC, num_subcores=_NS)

    @functools.partial(
        pl.kernel,
        out_type=jax.ShapeDtypeStruct((n, _D), jnp.float32),
        mesh=mesh,
        scratch_types=[
            pltpu.VMEM((per_w,), jnp.int32),
            pltpu.VMEM((_NBUF, _CHUNK, _D), jnp.float32),
            pltpu.SemaphoreType.DMA,
            pltpu.SemaphoreType.DMA,
            pltpu.SemaphoreType.DMA,
            pltpu.SemaphoreType.DMA,
        ],
        compiler_params=pltpu.CompilerParams(use_tc_tiling_on_sc=False),
    )
    def emb(idx_hbm, table_hbm, out_hbm, idx_all, rows, sg0, sg1, so0, so1):
        sg = (sg0, sg1)
        so = (so0, so1)
        wid = lax.axis_index("s") * _NC + lax.axis_index("c")
        w_base = wid * per_w
        pltpu.sync_copy(idx_hbm.at[pl.ds(w_base, per_w)], idx_all)

        def gather_desc(c, b):
            return pltpu.make_async_copy(
                table_hbm.at[idx_all.at[pl.ds(c * _CHUNK, _CHUNK)]],
                rows.at[b], sg[b])

        def store_desc(c, b):
            return pltpu.make_async_copy(
                rows.at[b], out_hbm.at[pl.ds(w_base + c * _CHUNK, _CHUNK)],
                so[b])

        def scale(b):
            def scale_row(i, carry):
                rows[b, i, pl.ds(0, _L)] = rows[b, i, pl.ds(0, _L)] * _SCALE
                rows[b, i, pl.ds(_L, _L)] = rows[b, i, pl.ds(_L, _L)] * _SCALE
                return carry
            lax.fori_loop(0, _CHUNK, scale_row, 0)

        for b in range(_NBUF):
            gather_desc(b, b).start()

        @pl.loop(0, nsuper - 1)
        def super_step(g):
            for b in range(_NBUF):
                c = g * _NBUF + b
                gather_desc(c, b).wait()
                scale(b)
                sd = store_desc(c, b)
                sd.start()
                sd.wait()
                gather_desc(c + _NBUF, b).start()

        for b in range(_NBUF):
            c_last = (nsuper - 1) * _NBUF + b
            gather_desc(c_last, b).wait()
            scale(b)
            store_desc(c_last, b).start()
        for b in range(_NBUF):
            store_desc(nchunk - _NBUF + b, b).wait()

    out = emb(idx_flat, embedding_table)
    return out.reshape(batch, seqlen, _D)
